# baseline (speedup 1.0000x reference)
import jax
import jax.numpy as jnp
from jax import lax
from jax.experimental import pallas as pl
from jax.experimental.pallas import tpu as pltpu
from jax.experimental.pallas import tpu_sc as plsc

N = 50000
D = 128
E = 200000
NUM_REL = 3

NC = 2
NS = 16
G = 112
NBANK = 4
CP = 4
CW = 32

E_PAD = 200704
EPT = E_PAD // NS
RI = EPT // G
NB = RI
CHK = 28
NCHK = NB // CHK
N_PAD = 50176
RPT = N_PAD // NS
ZR = 112
HR = RPT // 2
TB = 400


def _sc_body(xv, srcs, dsts, sums, cnts,
             accum, cs0, cd0, cs1, cd1, rows_a, rows_b, rows_c, rows_d, zbuf,
             sem_ga, sem_gb, sem_gc, sem_gd,
             sem_sa, sem_sb, sem_sc, sem_sd, sem_z, sem_ia, sem_ib):
    c = lax.axis_index("c")
    t = lax.axis_index("s")

    @pl.loop(0, ZR)
    def _(i):
        zbuf[i, pl.ds(0, 16)] = jnp.zeros((16,), jnp.float32)
        zbuf[i, pl.ds(16, 16)] = jnp.zeros((16,), jnp.float32)

    def zero_own_slice():
        @pl.loop(0, RPT // ZR)
        def _(k):
            pltpu.make_async_copy(
                zbuf, accum.at[pl.ds(t * RPT + k * ZR, ZR)], sem_z).start()

        @pl.loop(0, RPT // ZR)
        def _(k):
            pltpu.make_async_copy(
                zbuf, accum.at[pl.ds(t * RPT + k * ZR, ZR)], sem_z).wait()

    zero_own_slice()
    plsc.subcore_barrier()

    banks = [(rows_a, sem_ga, sem_sa), (rows_b, sem_gb, sem_sb),
             (rows_c, sem_gc, sem_sc), (rows_d, sem_gd, sem_sd)]

    def fire_g(cs, bank, lrow):
        rows, gsem, _ = banks[bank]
        pltpu.make_async_copy(xv.at[cs.at[lrow]], rows, gsem).start()

    def wait_g(cs, bank, lrow):
        rows, gsem, _ = banks[bank]
        pltpu.make_async_copy(xv.at[cs.at[lrow]], rows, gsem).wait()

    def fire_s(cd, bank, lrow):
        rows, _, ssem = banks[bank]
        pltpu.make_async_copy(rows, accum.at[cd.at[lrow]],
                              ssem).start(add=True)

    def drain_s(cd, bank, lrow):
        rows, _, ssem = banks[bank]
        pltpu.make_async_copy(rows, accum.at[cd.at[lrow]], ssem).wait()

    def idx_copies(cs, cd, isem, r, p, ck):
        return (pltpu.make_async_copy(srcs.at[r, p, t, pl.ds(ck * CHK, CHK)],
                                      cs, isem),
                pltpu.make_async_copy(dsts.at[r, t, pl.ds(ck * CHK, CHK)],
                                      cd, isem))

    def process_chunk(cs, cd):
        fire_g(cs, 0, 0)
        fire_g(cs, 1, 1)
        fire_g(cs, 2, 2)
        wait_g(cs, 0, 0)
        fire_s(cd, 0, 0)
        fire_g(cs, 3, 3)
        for i in range(1, NBANK):
            wait_g(cs, i, i)
            fire_s(cd, i, i)
            drain_s(cd, i - 1, i - 1)
            fire_g(cs, i - 1, i + 3)

        @pl.loop(1, CHK // NBANK - 1)
        def _(g):
            for i in range(NBANK):
                r_ = NBANK * g + i
                wait_g(cs, i, r_)
                fire_s(cd, i, r_)
                drain_s(cd, (i + 3) % NBANK, r_ - 1)
                fire_g(cs, (i + 3) % NBANK, r_ + 3)

        kt = CHK - NBANK
        wait_g(cs, 0, kt)
        fire_s(cd, 0, kt)
        drain_s(cd, 3, kt - 1)
        fire_g(cs, 3, kt + 3)
        for i in range(1, NBANK):
            wait_g(cs, i, kt + i)
            fire_s(cd, i, kt + i)
            drain_s(cd, i - 1, kt + i - 1)
        drain_s(cd, 3, kt + 3)

    def scatter_loop(r, p):
        for d_ in idx_copies(cs0, cd0, sem_ia, r, p, 0):
            d_.start()

        @pl.loop(0, NCHK // 2)
        def _(h):
            for d_ in idx_copies(cs0, cd0, sem_ia, r, p, 2 * h):
                d_.wait()
            for d_ in idx_copies(cs1, cd1, sem_ib, r, p, 2 * h + 1):
                d_.start()
            process_chunk(cs0, cd0)
            for d_ in idx_copies(cs1, cd1, sem_ib, r, p, 2 * h + 1):
                d_.wait()

            @pl.when(h < NCHK // 2 - 1)
            def _():
                for d_ in idx_copies(cs0, cd0, sem_ia, r, p, 2 * h + 2):
                    d_.start()

            process_chunk(cs1, cd1)

    def count_loop(r, ck_lo, ck_hi):
        @pl.loop(0, G)
        def _(i):
            rows_a[i, pl.ds(0, 16)] = jnp.ones((16,), jnp.float32)
            rows_a[i, pl.ds(16, 16)] = jnp.ones((16,), jnp.float32)

        @pl.loop(ck_lo, ck_hi)
        def _(ck):
            pltpu.sync_copy(dsts.at[r, t, pl.ds(ck * CHK, CHK)], cd0)

            @pl.loop(0, CHK)
            def _(i):
                pltpu.make_async_copy(rows_a, accum.at[cd0.at[i]],
                                      sem_sa).start(add=True)

            @pl.loop(0, CHK)
            def _(i):
                pltpu.make_async_copy(rows_a, accum.at[cd0.at[i]],
                                      sem_sa).wait()

    def wb_zero(d0, d1):
        w0 = pltpu.make_async_copy(accum.at[pl.ds(t * RPT, HR)], d0, sem_ia)
        w0.start()
        w1 = pltpu.make_async_copy(accum.at[pl.ds(t * RPT + HR, HR)], d1,
                                   sem_ib)
        w1.start()
        w0.wait()

        @pl.loop(0, HR // ZR)
        def _(k):
            pltpu.make_async_copy(
                zbuf, accum.at[pl.ds(t * RPT + k * ZR, ZR)], sem_z).start()

        w1.wait()

        @pl.loop(0, HR // ZR)
        def _(k):
            pltpu.make_async_copy(
                zbuf, accum.at[pl.ds(t * RPT + HR + k * ZR, ZR)],
                sem_z).start()

        @pl.loop(0, RPT // ZR)
        def _(k):
            pltpu.make_async_copy(
                zbuf, accum.at[pl.ds(t * RPT + k * ZR, ZR)], sem_z).wait()

    @pl.loop(0, NUM_REL * (CP // 2))
    def _(q):
        r = q // 2
        p = q % 2 + (CP // 2) * c
        po = pl.multiple_of(p * CW, CW)
        scatter_loop(r, p)
        plsc.subcore_barrier()
        wb_zero(sums.at[r, pl.ds(t * RPT, HR), pl.ds(po, CW)],
                sums.at[r, pl.ds(t * RPT + HR, HR), pl.ds(po, CW)])
        plsc.subcore_barrier()

    @pl.loop(0, 2)
    def _(q):
        solo = q == 0
        r = jnp.where(solo, jnp.where(c == 0, 0, 2), 1)
        slot = jnp.where(solo, jnp.where(c == 0, 0, 2),
                         jnp.where(c == 0, 1, 3))
        ck_lo = jnp.where(solo, 0, (NCHK // 2) * c)
        ck_hi = jnp.where(solo, NCHK, (NCHK // 2) * (c + 1))
        count_loop(r, ck_lo, ck_hi)
        plsc.subcore_barrier()
        so = pl.multiple_of(slot * CW, CW)
        wb_zero(cnts.at[pl.ds(t * RPT, HR), pl.ds(so, CW)],
                cnts.at[pl.ds(t * RPT + HR, HR), pl.ds(so, CW)])
        plsc.subcore_barrier()


_sc_call = pl.kernel(
    _sc_body,
    out_type=[
        jax.ShapeDtypeStruct((NUM_REL, N_PAD, D), jnp.float32),
        jax.ShapeDtypeStruct((N_PAD, D), jnp.float32),
    ],
    mesh=plsc.VectorSubcoreMesh(core_axis_name="c", subcore_axis_name="s"),
    scratch_types=[
        pltpu.MemorySpace.VMEM_SHARED((N_PAD, CW), jnp.float32),
        pltpu.VMEM((CHK, G), jnp.int32),
        pltpu.VMEM((CHK, G), jnp.int32),
        pltpu.VMEM((CHK, G), jnp.int32),
        pltpu.VMEM((CHK, G), jnp.int32),
        pltpu.VMEM((G, CW), jnp.float32),
        pltpu.VMEM((G, CW), jnp.float32),
        pltpu.VMEM((G, CW), jnp.float32),
        pltpu.VMEM((G, CW), jnp.float32),
        pltpu.VMEM((ZR, CW), jnp.float32),
        pltpu.SemaphoreType.DMA,
        pltpu.SemaphoreType.DMA,
        pltpu.SemaphoreType.DMA,
        pltpu.SemaphoreType.DMA,
        pltpu.SemaphoreType.DMA,
        pltpu.SemaphoreType.DMA,
        pltpu.SemaphoreType.DMA,
        pltpu.SemaphoreType.DMA,
        pltpu.SemaphoreType.DMA,
        pltpu.SemaphoreType.DMA,
        pltpu.SemaphoreType.DMA,
    ],
    compiler_params=pltpu.CompilerParams(use_tc_tiling_on_sc=False),
)


def _tc_body(sums_ref, cnts_ref, x_ref, wl_ref, wr_ref, b_ref, o_ref):
    x = x_ref[...]
    wr = wr_ref[0] + wr_ref[1] + wr_ref[2]
    bias = b_ref[0] + b_ref[1] + b_ref[2]
    acc = lax.dot_general(x, wr, (((1,), (1,)), ((), ())),
                          preferred_element_type=jnp.float32) + bias[None, :]
    cn = cnts_ref[...]
    cnt_r = [cn[:, 0:1], cn[:, CW:CW + 1] + cn[:, 3 * CW:3 * CW + 1],
             cn[:, 2 * CW:2 * CW + 1]]
    for r in range(NUM_REL):
        mean = sums_ref[r] / jnp.maximum(cnt_r[r], 1.0)
        acc = acc + lax.dot_general(mean, wl_ref[r], (((1,), (1,)), ((), ())),
                                    preferred_element_type=jnp.float32)
    o_ref[...] = acc


def _tc_call(sums, cnts, x, wl, wr, bb):
    return pl.pallas_call(
        _tc_body,
        grid=(N // TB,),
        in_specs=[
            pl.BlockSpec((NUM_REL, TB, D), lambda i: (0, i, 0)),
            pl.BlockSpec((TB, D), lambda i: (i, 0)),
            pl.BlockSpec((TB, D), lambda i: (i, 0)),
            pl.BlockSpec((NUM_REL, D, D), lambda i: (0, 0, 0)),
            pl.BlockSpec((NUM_REL, D, D), lambda i: (0, 0, 0)),
            pl.BlockSpec((NUM_REL, D), lambda i: (0, 0)),
        ],
        out_specs=pl.BlockSpec((TB, D), lambda i: (i, 0)),
        out_shape=jax.ShapeDtypeStruct((N, D), jnp.float32),
    )(sums, cnts, x, wl, wr, bb)


def kernel(x, edge_index_r0, edge_index_r1, edge_index_r2,
           W_l_r0, b_r0, W_r_r0,
           W_l_r1, b_r1, W_r_r1,
           W_l_r2, b_r2, W_r_r2):
    pad = E_PAD - E
    src_list = []
    dst_list = []
    for ei in (edge_index_r0, edge_index_r1, edge_index_r2):
        src = jnp.concatenate([ei[0], jnp.zeros((pad,), jnp.int32)])
        src_list.append((src[None, :] * CP
                         + jnp.arange(CP, dtype=jnp.int32)[:, None]
                         ).reshape(CP, NS, RI, G))
        dst = jnp.concatenate([ei[1], jnp.full((pad,), N, jnp.int32)])
        dst_list.append(dst.reshape(NS, RI, G))
    srcs = jnp.stack(src_list)
    dsts = jnp.stack(dst_list)
    xv = x.reshape(CP * N, CW)
    wl = jnp.stack([W_l_r0, W_l_r1, W_l_r2])
    wr = jnp.stack([W_r_r0, W_r_r1, W_r_r2])
    bb = jnp.stack([b_r0, b_r1, b_r2])
    sums, cnts = _sc_call(xv, srcs, dsts)
    return _tc_call(sums, cnts, x, wl, wr, bb)


# speedup vs baseline: 1.0390x; 1.0390x over previous
import jax
import jax.numpy as jnp
from jax import lax
from jax.experimental import pallas as pl
from jax.experimental.pallas import tpu as pltpu
from jax.experimental.pallas import tpu_sc as plsc

N = 50000
D = 128
E = 200000
NUM_REL = 3

NC = 2
NS = 16
G = 112
NBANK = 4
CP = 4
CW = 32

E_PAD = 200704
EPT = E_PAD // NS
RI = EPT // G
NB = RI
CHK = 28
NCHK = NB // CHK
N_PAD = 50176
RPT = N_PAD // NS
ZR = 112
HR = RPT // 2
TB = 1024


def _sc_body(xv, srcs, dsts, sums, cnts,
             accum, cs0, cd0, cs1, cd1, rows_a, rows_b, rows_c, rows_d, zbuf,
             sem_ga, sem_gb, sem_gc, sem_gd,
             sem_sa, sem_sb, sem_sc, sem_sd, sem_z, sem_ia, sem_ib):
    c = lax.axis_index("c")
    t = lax.axis_index("s")

    @pl.loop(0, ZR)
    def _(i):
        zbuf[i, pl.ds(0, 16)] = jnp.zeros((16,), jnp.float32)
        zbuf[i, pl.ds(16, 16)] = jnp.zeros((16,), jnp.float32)

    def zero_own_slice():
        @pl.loop(0, RPT // ZR)
        def _(k):
            pltpu.make_async_copy(
                zbuf, accum.at[pl.ds(t * RPT + k * ZR, ZR)], sem_z).start()

        @pl.loop(0, RPT // ZR)
        def _(k):
            pltpu.make_async_copy(
                zbuf, accum.at[pl.ds(t * RPT + k * ZR, ZR)], sem_z).wait()

    zero_own_slice()
    plsc.subcore_barrier()

    banks = [(rows_a, sem_ga, sem_sa), (rows_b, sem_gb, sem_sb),
             (rows_c, sem_gc, sem_sc), (rows_d, sem_gd, sem_sd)]

    def fire_g(cs, bank, lrow):
        rows, gsem, _ = banks[bank]
        pltpu.make_async_copy(xv.at[cs.at[lrow]], rows, gsem).start()

    def wait_g(cs, bank, lrow):
        rows, gsem, _ = banks[bank]
        pltpu.make_async_copy(xv.at[cs.at[lrow]], rows, gsem).wait()

    def fire_s(cd, bank, lrow):
        rows, _, ssem = banks[bank]
        pltpu.make_async_copy(rows, accum.at[cd.at[lrow]],
                              ssem).start(add=True)

    def drain_s(cd, bank, lrow):
        rows, _, ssem = banks[bank]
        pltpu.make_async_copy(rows, accum.at[cd.at[lrow]], ssem).wait()

    def idx_copies(cs, cd, isem, r, p, ck):
        return (pltpu.make_async_copy(srcs.at[r, p, t, pl.ds(ck * CHK, CHK)],
                                      cs, isem),
                pltpu.make_async_copy(dsts.at[r, t, pl.ds(ck * CHK, CHK)],
                                      cd, isem))

    def process_chunk(cs, cd):
        fire_g(cs, 0, 0)
        fire_g(cs, 1, 1)
        fire_g(cs, 2, 2)
        wait_g(cs, 0, 0)
        fire_s(cd, 0, 0)
        fire_g(cs, 3, 3)
        for i in range(1, NBANK):
            wait_g(cs, i, i)
            fire_s(cd, i, i)
            drain_s(cd, i - 1, i - 1)
            fire_g(cs, i - 1, i + 3)

        @pl.loop(1, CHK // NBANK - 1)
        def _(g):
            for i in range(NBANK):
                r_ = NBANK * g + i
                wait_g(cs, i, r_)
                fire_s(cd, i, r_)
                drain_s(cd, (i + 3) % NBANK, r_ - 1)
                fire_g(cs, (i + 3) % NBANK, r_ + 3)

        kt = CHK - NBANK
        wait_g(cs, 0, kt)
        fire_s(cd, 0, kt)
        drain_s(cd, 3, kt - 1)
        fire_g(cs, 3, kt + 3)
        for i in range(1, NBANK):
            wait_g(cs, i, kt + i)
            fire_s(cd, i, kt + i)
            drain_s(cd, i - 1, kt + i - 1)
        drain_s(cd, 3, kt + 3)

    def scatter_loop(r, p):
        for d_ in idx_copies(cs0, cd0, sem_ia, r, p, 0):
            d_.start()

        @pl.loop(0, NCHK // 2)
        def _(h):
            for d_ in idx_copies(cs0, cd0, sem_ia, r, p, 2 * h):
                d_.wait()
            for d_ in idx_copies(cs1, cd1, sem_ib, r, p, 2 * h + 1):
                d_.start()
            process_chunk(cs0, cd0)
            for d_ in idx_copies(cs1, cd1, sem_ib, r, p, 2 * h + 1):
                d_.wait()

            @pl.when(h < NCHK // 2 - 1)
            def _():
                for d_ in idx_copies(cs0, cd0, sem_ia, r, p, 2 * h + 2):
                    d_.start()

            process_chunk(cs1, cd1)

    def count_loop(r, ck_lo, ck_hi):
        @pl.loop(0, G)
        def _(i):
            rows_a[i, pl.ds(0, 16)] = jnp.ones((16,), jnp.float32)
            rows_a[i, pl.ds(16, 16)] = jnp.ones((16,), jnp.float32)

        @pl.loop(ck_lo, ck_hi)
        def _(ck):
            pltpu.sync_copy(dsts.at[r, t, pl.ds(ck * CHK, CHK)], cd0)

            @pl.loop(0, CHK)
            def _(i):
                pltpu.make_async_copy(rows_a, accum.at[cd0.at[i]],
                                      sem_sa).start(add=True)

            @pl.loop(0, CHK)
            def _(i):
                pltpu.make_async_copy(rows_a, accum.at[cd0.at[i]],
                                      sem_sa).wait()

    def wb_zero(d0, d1):
        w0 = pltpu.make_async_copy(accum.at[pl.ds(t * RPT, HR)], d0, sem_ia)
        w0.start()
        w1 = pltpu.make_async_copy(accum.at[pl.ds(t * RPT + HR, HR)], d1,
                                   sem_ib)
        w1.start()
        w0.wait()

        @pl.loop(0, HR // ZR)
        def _(k):
            pltpu.make_async_copy(
                zbuf, accum.at[pl.ds(t * RPT + k * ZR, ZR)], sem_z).start()

        w1.wait()

        @pl.loop(0, HR // ZR)
        def _(k):
            pltpu.make_async_copy(
                zbuf, accum.at[pl.ds(t * RPT + HR + k * ZR, ZR)],
                sem_z).start()

        @pl.loop(0, RPT // ZR)
        def _(k):
            pltpu.make_async_copy(
                zbuf, accum.at[pl.ds(t * RPT + k * ZR, ZR)], sem_z).wait()

    @pl.loop(0, NUM_REL * (CP // 2))
    def _(q):
        r = q // 2
        p = q % 2 + (CP // 2) * c
        po = pl.multiple_of(p * CW, CW)
        scatter_loop(r, p)
        plsc.subcore_barrier()
        wb_zero(sums.at[r, pl.ds(t * RPT, HR), pl.ds(po, CW)],
                sums.at[r, pl.ds(t * RPT + HR, HR), pl.ds(po, CW)])
        plsc.subcore_barrier()

    @pl.loop(0, 2)
    def _(q):
        solo = q == 0
        r = jnp.where(solo, jnp.where(c == 0, 0, 2), 1)
        slot = jnp.where(solo, jnp.where(c == 0, 0, 2),
                         jnp.where(c == 0, 1, 3))
        ck_lo = jnp.where(solo, 0, (NCHK // 2) * c)
        ck_hi = jnp.where(solo, NCHK, (NCHK // 2) * (c + 1))
        count_loop(r, ck_lo, ck_hi)
        plsc.subcore_barrier()
        so = pl.multiple_of(slot * CW, CW)
        wb_zero(cnts.at[pl.ds(t * RPT, HR), pl.ds(so, CW)],
                cnts.at[pl.ds(t * RPT + HR, HR), pl.ds(so, CW)])
        plsc.subcore_barrier()


_sc_call = pl.kernel(
    _sc_body,
    out_type=[
        jax.ShapeDtypeStruct((NUM_REL, N_PAD, D), jnp.float32),
        jax.ShapeDtypeStruct((N_PAD, D), jnp.float32),
    ],
    mesh=plsc.VectorSubcoreMesh(core_axis_name="c", subcore_axis_name="s"),
    scratch_types=[
        pltpu.MemorySpace.VMEM_SHARED((N_PAD, CW), jnp.float32),
        pltpu.VMEM((CHK, G), jnp.int32),
        pltpu.VMEM((CHK, G), jnp.int32),
        pltpu.VMEM((CHK, G), jnp.int32),
        pltpu.VMEM((CHK, G), jnp.int32),
        pltpu.VMEM((G, CW), jnp.float32),
        pltpu.VMEM((G, CW), jnp.float32),
        pltpu.VMEM((G, CW), jnp.float32),
        pltpu.VMEM((G, CW), jnp.float32),
        pltpu.VMEM((ZR, CW), jnp.float32),
        pltpu.SemaphoreType.DMA,
        pltpu.SemaphoreType.DMA,
        pltpu.SemaphoreType.DMA,
        pltpu.SemaphoreType.DMA,
        pltpu.SemaphoreType.DMA,
        pltpu.SemaphoreType.DMA,
        pltpu.SemaphoreType.DMA,
        pltpu.SemaphoreType.DMA,
        pltpu.SemaphoreType.DMA,
        pltpu.SemaphoreType.DMA,
        pltpu.SemaphoreType.DMA,
    ],
    compiler_params=pltpu.CompilerParams(use_tc_tiling_on_sc=False),
)


def _tc_body(sums_ref, cnts_ref, x_ref, wl_ref, wr_ref, b_ref, o_ref):
    x = x_ref[...]
    wr = wr_ref[0] + wr_ref[1] + wr_ref[2]
    bias = b_ref[0] + b_ref[1] + b_ref[2]
    acc = lax.dot_general(x, wr, (((1,), (1,)), ((), ())),
                          preferred_element_type=jnp.float32) + bias[None, :]
    cn = cnts_ref[...]
    cnt_r = [cn[:, 0:1], cn[:, CW:CW + 1] + cn[:, 3 * CW:3 * CW + 1],
             cn[:, 2 * CW:2 * CW + 1]]
    for r in range(NUM_REL):
        mean = sums_ref[r] / jnp.maximum(cnt_r[r], 1.0)
        acc = acc + lax.dot_general(mean, wl_ref[r], (((1,), (1,)), ((), ())),
                                    preferred_element_type=jnp.float32)
    o_ref[...] = acc


def _tc_call(sums, cnts, x, wl, wr, bb):
    return pl.pallas_call(
        _tc_body,
        grid=(N_PAD // TB,),
        in_specs=[
            pl.BlockSpec((NUM_REL, TB, D), lambda i: (0, i, 0)),
            pl.BlockSpec((TB, D), lambda i: (i, 0)),
            pl.BlockSpec((TB, D), lambda i: (i, 0)),
            pl.BlockSpec((NUM_REL, D, D), lambda i: (0, 0, 0)),
            pl.BlockSpec((NUM_REL, D, D), lambda i: (0, 0, 0)),
            pl.BlockSpec((NUM_REL, D), lambda i: (0, 0)),
        ],
        out_specs=pl.BlockSpec((TB, D), lambda i: (i, 0)),
        out_shape=jax.ShapeDtypeStruct((N_PAD, D), jnp.float32),
    )(sums, cnts, x, wl, wr, bb)


def kernel(x, edge_index_r0, edge_index_r1, edge_index_r2,
           W_l_r0, b_r0, W_r_r0,
           W_l_r1, b_r1, W_r_r1,
           W_l_r2, b_r2, W_r_r2):
    pad = E_PAD - E
    src_list = []
    dst_list = []
    for ei in (edge_index_r0, edge_index_r1, edge_index_r2):
        src = jnp.concatenate([ei[0], jnp.zeros((pad,), jnp.int32)])
        src_list.append((src[None, :] * CP
                         + jnp.arange(CP, dtype=jnp.int32)[:, None]
                         ).reshape(CP, NS, RI, G))
        dst = jnp.concatenate([ei[1], jnp.full((pad,), N, jnp.int32)])
        dst_list.append(dst.reshape(NS, RI, G))
    srcs = jnp.stack(src_list)
    dsts = jnp.stack(dst_list)
    xv = x.reshape(CP * N, CW)
    wl = jnp.stack([W_l_r0, W_l_r1, W_l_r2])
    wr = jnp.stack([W_r_r0, W_r_r1, W_r_r2])
    bb = jnp.stack([b_r0, b_r1, b_r2])
    sums, cnts = _sc_call(xv, srcs, dsts)
    x_pad = jnp.pad(x, ((0, N_PAD - N), (0, 0)))
    out = _tc_call(sums, cnts, x_pad, wl, wr, bb)
    return out[:N]
